# baseline (speedup 1.0000x reference)
import functools

import jax
import jax.numpy as jnp
from jax.experimental import pallas as pl
from jax.experimental.pallas import tpu as pltpu

BM = 2048
BK = 512
CHUNK_BOUNDS = (2736, 5472)


def _bf16_rne(x):
    u = jax.lax.bitcast_convert_type(x, jnp.uint32)
    u = (u + jnp.uint32(0x7FFF) + ((u >> 16) & jnp.uint32(1))) \
        & jnp.uint32(0xFFFF0000)
    return jax.lax.bitcast_convert_type(u, jnp.float32)


def _body(x_ref, xb_ref, ytb_ref, y2_ref, out_ref,
          vacc_ref, jacc_ref, cv_ref, ci_ref, x2_ref, *, nk, bk):
    j = pl.program_id(1)
    xw = jax.lax.dot_general(
        xb_ref[...], ytb_ref[...], (((1,), (0,)), ((), ())),
        preferred_element_type=jnp.float32)

    @pl.when(j == 0)
    def _x2():
        x = x_ref[...]
        x2_ref[...] = jnp.sum(x * x, axis=1, keepdims=True)

    y2 = y2_ref[...].reshape(1, bk)
    m = (x2_ref[...] + y2) - 2.0 * xw
    s = m * jax.lax.rsqrt(m)

    inf = jnp.float32(jnp.inf)
    big = jnp.int32(2**31 - 1)
    straddle_tiles = [cb // bk for cb in CHUNK_BOUNDS if cb % bk and cb < nk * bk]

    def lane_reduce(vacc, jacc):
        col = jax.lax.broadcasted_iota(jnp.int32, vacc.shape, 1)
        gidx = jacc.astype(jnp.int32) * bk + col
        mv = jnp.min(vacc, axis=1, keepdims=True)
        mi = jnp.min(jnp.where(vacc == mv, gidx, big), axis=1, keepdims=True)
        return mv, mi

    @pl.when(j == 0)
    def _init():
        vacc_ref[...] = s
        jacc_ref[...] = jnp.zeros(jacc_ref.shape, jacc_ref.dtype)
        cv_ref[...] = jnp.full(cv_ref.shape, inf, jnp.float32)
        ci_ref[...] = jnp.zeros(ci_ref.shape, jnp.int32)

    is_straddle = (j == straddle_tiles[0]) | (j == straddle_tiles[1]) \
        if len(straddle_tiles) == 2 else (j < 0)

    @pl.when((j > 0) & jnp.logical_not(is_straddle))
    def _update():
        vacc = vacc_ref[...]
        upd = s < vacc
        vacc_ref[...] = jnp.where(upd, s, vacc)
        jacc_ref[...] = jnp.where(upd, jnp.array(0, jacc_ref.dtype) + j,
                                  jacc_ref[...])

    for cb in CHUNK_BOUNDS:
        if cb % bk == 0 or cb >= nk * bk:
            continue

        @pl.when(j == cb // bk)
        def _chunk_end(cb=cb):
            gcol = jax.lax.broadcasted_iota(jnp.int32, s.shape, 1) + j * bk
            sp = jnp.where(gcol < cb, s, inf)
            vacc = vacc_ref[...]
            upd = sp < vacc
            vacc = jnp.where(upd, sp, vacc)
            jacc = jnp.where(upd, jnp.array(0, jacc_ref.dtype) + j,
                             jacc_ref[...])
            mv, mi = lane_reduce(vacc, jacc)
            cv, ci = cv_ref[...], ci_ref[...]
            keep = cv <= mv
            cv_ref[...] = _bf16_rne(jnp.where(keep, cv, mv))
            ci_ref[...] = jnp.where(keep, ci, mi)
            vacc_ref[...] = jnp.where(gcol >= cb, s, inf)
            jacc_ref[...] = jnp.full(jacc_ref.shape, j, jacc_ref.dtype)

    @pl.when(j == nk - 1)
    def _emit():
        mv, mi = lane_reduce(vacc_ref[...], jacc_ref[...])
        cv, ci = cv_ref[...], ci_ref[...]
        keep = cv <= mv
        out_ref[...] = jnp.where(keep, ci, mi)


def kernel(embeddings, table):
    B, S, D = embeddings.shape
    K = table.shape[0]
    N = B * S
    bm = min(BM, N)
    bk = min(BK, K)
    nk = K // bk
    flat = embeddings.reshape(N, D)
    yt = table.T
    xb = flat.astype(jnp.bfloat16)
    ytb = yt.astype(jnp.bfloat16)
    y2 = jnp.sum(table * table, axis=-1)
    out = pl.pallas_call(
        functools.partial(_body, nk=nk, bk=bk),
        grid=(N // bm, nk),
        in_specs=[
            pl.BlockSpec((bm, D), lambda i, j: (i, 0)),
            pl.BlockSpec((bm, D), lambda i, j: (i, 0)),
            pl.BlockSpec((D, bk), lambda i, j: (0, j)),
            pl.BlockSpec((bk,), lambda i, j: (j,)),
        ],
        out_specs=pl.BlockSpec((bm, 1), lambda i, j: (i, 0)),
        out_shape=jax.ShapeDtypeStruct((N, 1), jnp.int32),
        scratch_shapes=[
            pltpu.VMEM((bm, bk), jnp.float32),
            pltpu.VMEM((bm, bk), jnp.int32),
            pltpu.VMEM((bm, 1), jnp.float32),
            pltpu.VMEM((bm, 1), jnp.int32),
            pltpu.VMEM((bm, 1), jnp.float32),
        ],
    )(flat, xb, ytb, y2)
    return out.reshape(B, S)


# speedup vs baseline: 1.0234x; 1.0234x over previous
import functools

import jax
import jax.numpy as jnp
from jax.experimental import pallas as pl
from jax.experimental.pallas import tpu as pltpu

BM = 2048
BK = 512
CHUNK_BOUNDS = (2736, 5472)


def _bf16_rne(x):
    u = jax.lax.bitcast_convert_type(x, jnp.uint32)
    u = (u + jnp.uint32(0x7FFF) + ((u >> 16) & jnp.uint32(1))) \
        & jnp.uint32(0xFFFF0000)
    return jax.lax.bitcast_convert_type(u, jnp.float32)


def _body(x_ref, yt_ref, out_ref,
          vacc_ref, jacc_ref, cv_ref, ci_ref, x2_ref, *, nk, bk):
    j = pl.program_id(1)
    yt = yt_ref[...]
    xw = jax.lax.dot_general(
        x_ref[...].astype(jnp.bfloat16), yt.astype(jnp.bfloat16),
        (((1,), (0,)), ((), ())),
        preferred_element_type=jnp.float32)

    @pl.when(j == 0)
    def _x2():
        x = x_ref[...]
        x2_ref[...] = jnp.sum(x * x, axis=1, keepdims=True)

    y2 = jnp.sum(yt * yt, axis=0, keepdims=True)
    m = (x2_ref[...] + y2) - 2.0 * xw
    s = m * jax.lax.rsqrt(m)

    inf = jnp.float32(jnp.inf)
    big = jnp.int32(2**31 - 1)
    straddle_tiles = [cb // bk for cb in CHUNK_BOUNDS if cb % bk and cb < nk * bk]

    def lane_reduce(vacc, jacc):
        col = jax.lax.broadcasted_iota(jnp.int32, vacc.shape, 1)
        gidx = jacc.astype(jnp.int32) * bk + col
        mv = jnp.min(vacc, axis=1, keepdims=True)
        mi = jnp.min(jnp.where(vacc == mv, gidx, big), axis=1, keepdims=True)
        return mv, mi

    @pl.when(j == 0)
    def _init():
        vacc_ref[...] = s
        jacc_ref[...] = jnp.zeros(jacc_ref.shape, jacc_ref.dtype)
        cv_ref[...] = jnp.full(cv_ref.shape, inf, jnp.float32)
        ci_ref[...] = jnp.zeros(ci_ref.shape, jnp.int32)

    is_straddle = (j == straddle_tiles[0]) | (j == straddle_tiles[1]) \
        if len(straddle_tiles) == 2 else (j < 0)

    @pl.when((j > 0) & jnp.logical_not(is_straddle))
    def _update():
        vacc = vacc_ref[...]
        upd = s < vacc
        vacc_ref[...] = jnp.where(upd, s, vacc)
        jacc_ref[...] = jnp.where(upd, jnp.array(0, jacc_ref.dtype) + j,
                                  jacc_ref[...])

    for cb in CHUNK_BOUNDS:
        if cb % bk == 0 or cb >= nk * bk:
            continue

        @pl.when(j == cb // bk)
        def _chunk_end(cb=cb):
            gcol = jax.lax.broadcasted_iota(jnp.int32, s.shape, 1) + j * bk
            sp = jnp.where(gcol < cb, s, inf)
            vacc = vacc_ref[...]
            upd = sp < vacc
            vacc = jnp.where(upd, sp, vacc)
            jacc = jnp.where(upd, jnp.array(0, jacc_ref.dtype) + j,
                             jacc_ref[...])
            mv, mi = lane_reduce(vacc, jacc)
            cv, ci = cv_ref[...], ci_ref[...]
            keep = cv <= mv
            cv_ref[...] = _bf16_rne(jnp.where(keep, cv, mv))
            ci_ref[...] = jnp.where(keep, ci, mi)
            vacc_ref[...] = jnp.where(gcol >= cb, s, inf)
            jacc_ref[...] = jnp.full(jacc_ref.shape, j, jacc_ref.dtype)

    @pl.when(j == nk - 1)
    def _emit():
        mv, mi = lane_reduce(vacc_ref[...], jacc_ref[...])
        cv, ci = cv_ref[...], ci_ref[...]
        keep = cv <= mv
        out_ref[...] = jnp.where(keep, ci, mi)


def kernel(embeddings, table):
    B, S, D = embeddings.shape
    K = table.shape[0]
    N = B * S
    bm = min(BM, N)
    bk = min(BK, K)
    nk = K // bk
    flat = embeddings.reshape(N, D)
    yt = table.T
    out = pl.pallas_call(
        functools.partial(_body, nk=nk, bk=bk),
        grid=(N // bm, nk),
        in_specs=[
            pl.BlockSpec((bm, D), lambda i, j: (i, 0)),
            pl.BlockSpec((D, bk), lambda i, j: (0, j)),
        ],
        out_specs=pl.BlockSpec((bm, 1), lambda i, j: (i, 0)),
        out_shape=jax.ShapeDtypeStruct((N, 1), jnp.int32),
        scratch_shapes=[
            pltpu.VMEM((bm, bk), jnp.float32),
            pltpu.VMEM((bm, bk), jnp.int32),
            pltpu.VMEM((bm, 1), jnp.float32),
            pltpu.VMEM((bm, 1), jnp.int32),
            pltpu.VMEM((bm, 1), jnp.float32),
        ],
    )(flat, yt)
    return out.reshape(B, S)


# speedup vs baseline: 1.1721x; 1.1453x over previous
import functools

import jax
import jax.numpy as jnp
from jax.experimental import pallas as pl
from jax.experimental.pallas import tpu as pltpu

BM = 2048
BK = 512
CHUNK_BOUNDS = (2736, 5472)


def _bf16_rne(x):
    u = jax.lax.bitcast_convert_type(x, jnp.uint32)
    u = (u + jnp.uint32(0x7FFF) + ((u >> 16) & jnp.uint32(1))) \
        & jnp.uint32(0xFFFF0000)
    return jax.lax.bitcast_convert_type(u, jnp.float32)


def _body(x_ref, yt_ref, out_ref,
          vacc_ref, jacc_ref, cv_ref, ci_ref, *, nk, bk):
    j = pl.program_id(1)
    x = x_ref[...]
    yt = yt_ref[...]
    xw = jax.lax.dot_general(
        x.astype(jnp.bfloat16), yt.astype(jnp.bfloat16),
        (((1,), (0,)), ((), ())),
        preferred_element_type=jnp.float32)
    x2 = jnp.sum(x * x, axis=1, keepdims=True)
    y2 = jnp.sum(yt * yt, axis=0, keepdims=True)
    m = jnp.maximum((x2 + y2) - 2.0 * xw, 1e-12)
    s = m * jax.lax.rsqrt(m)

    inf = jnp.float32(jnp.inf)
    big = jnp.int32(2**31 - 1)
    straddle_tiles = [cb // bk for cb in CHUNK_BOUNDS if cb % bk and cb < nk * bk]

    def lane_reduce(vacc, jacc):
        col = jax.lax.broadcasted_iota(jnp.int32, vacc.shape, 1)
        gidx = jacc.astype(jnp.int32) * bk + col
        mv = jnp.min(vacc, axis=1, keepdims=True)
        mi = jnp.min(jnp.where(vacc == mv, gidx, big), axis=1, keepdims=True)
        return mv, mi

    @pl.when(j == 0)
    def _init():
        vacc_ref[...] = s
        jacc_ref[...] = jnp.zeros(jacc_ref.shape, jacc_ref.dtype)
        cv_ref[...] = jnp.full(cv_ref.shape, inf, jnp.float32)
        ci_ref[...] = jnp.zeros(ci_ref.shape, jnp.int32)

    is_straddle = (j == straddle_tiles[0]) | (j == straddle_tiles[1]) \
        if len(straddle_tiles) == 2 else (j < 0)

    @pl.when((j > 0) & jnp.logical_not(is_straddle))
    def _update():
        vacc = vacc_ref[...]
        upd = s < vacc
        vacc_ref[...] = jnp.where(upd, s, vacc)
        jacc_ref[...] = jnp.where(upd, jnp.array(0, jacc_ref.dtype) + j,
                                  jacc_ref[...])

    for cb in CHUNK_BOUNDS:
        if cb % bk == 0 or cb >= nk * bk:
            continue

        @pl.when(j == cb // bk)
        def _chunk_end(cb=cb):
            gcol = jax.lax.broadcasted_iota(jnp.int32, s.shape, 1) + j * bk
            sp = jnp.where(gcol < cb, s, inf)
            vacc = vacc_ref[...]
            upd = sp < vacc
            vacc = jnp.where(upd, sp, vacc)
            jacc = jnp.where(upd, jnp.array(0, jacc_ref.dtype) + j,
                             jacc_ref[...])
            mv, mi = lane_reduce(vacc, jacc)
            cv, ci = cv_ref[...], ci_ref[...]
            keep = cv <= mv
            cv_ref[...] = _bf16_rne(jnp.where(keep, cv, mv))
            ci_ref[...] = jnp.where(keep, ci, mi)
            vacc_ref[...] = jnp.where(gcol >= cb, s, inf)
            jacc_ref[...] = jnp.full(jacc_ref.shape, j, jacc_ref.dtype)

    @pl.when(j == nk - 1)
    def _emit():
        mv, mi = lane_reduce(vacc_ref[...], jacc_ref[...])
        cv, ci = cv_ref[...], ci_ref[...]
        keep = cv <= mv
        out_ref[...] = jnp.where(keep, ci, mi)


def kernel(embeddings, table):
    B, S, D = embeddings.shape
    K = table.shape[0]
    N = B * S
    bm = min(BM, N)
    bk = min(BK, K)
    nk = K // bk
    flat = embeddings.reshape(N, D)
    yt = table.T
    out = pl.pallas_call(
        functools.partial(_body, nk=nk, bk=bk),
        grid=(N // bm, nk),
        in_specs=[
            pl.BlockSpec((bm, D), lambda i, j: (i, 0)),
            pl.BlockSpec((D, bk), lambda i, j: (0, j)),
        ],
        out_specs=pl.BlockSpec((bm, 1), lambda i, j: (i, 0)),
        out_shape=jax.ShapeDtypeStruct((N, 1), jnp.int32),
        scratch_shapes=[
            pltpu.VMEM((bm, bk), jnp.float32),
            pltpu.VMEM((bm, bk), jnp.int32),
            pltpu.VMEM((bm, 1), jnp.float32),
            pltpu.VMEM((bm, 1), jnp.int32),
        ],
    )(flat, yt)
    return out.reshape(B, S)
